# baseline (speedup 1.0000x reference)
import functools

import jax
import jax.numpy as jnp
from jax import lax
from jax.experimental import pallas as pl
from jax.experimental.pallas import tpu as pltpu
from jax.experimental.pallas import tpu_sc as plsc

N = 10000
D = 128
HID = 64
OUT = 7
G = 64
NEG = 0.2

FP = 128
DEN = HID
ASRC = HID + 1

NP_ = 10240
STRIPE = NP_ // 16
NC, NS = 2, 16
NW = NC * NS
GPT = 82
EPW = GPT * 128
ETOT = NW * EPW
PAD_ROWS = NP_ - N

_BLK = 1024


def _tc_embed_body(x_ref, w_ref, e64_ref, e65_ref, ats_ref, atd_ref,
                   xl_ref, ad_ref):
    xl = jnp.dot(x_ref[...], w_ref[...], preferred_element_type=jnp.float32)
    xl = xl + e64_ref[...][None, :]
    as_ = jnp.sum(xl * ats_ref[...][None, :], axis=1)
    ad_ref[...] = jnp.sum(xl * atd_ref[...][None, :], axis=1)
    xl_ref[...] = xl + as_[:, None] * e65_ref[...][None, :]


def _tc_embed(x_p, w_pad, e64, e65, ats_pad, atd_pad, d_in):
    return pl.pallas_call(
        _tc_embed_body,
        grid=(NP_ // _BLK,),
        in_specs=[
            pl.BlockSpec((_BLK, d_in), lambda i: (i, 0)),
            pl.BlockSpec((d_in, FP), lambda i: (0, 0)),
            pl.BlockSpec((FP,), lambda i: (0,)),
            pl.BlockSpec((FP,), lambda i: (0,)),
            pl.BlockSpec((FP,), lambda i: (0,)),
            pl.BlockSpec((FP,), lambda i: (0,)),
        ],
        out_specs=[
            pl.BlockSpec((_BLK, FP), lambda i: (i, 0)),
            pl.BlockSpec((_BLK,), lambda i: (i,)),
        ],
        out_shape=[
            jax.ShapeDtypeStruct((NP_, FP), jnp.float32),
            jax.ShapeDtypeStruct((NP_,), jnp.float32),
        ],
    )(x_p, w_pad, e64, e65, ats_pad, atd_pad)


def _tc_mid_body(o0_ref, o1_ref, b_ref, w_ref, e64_ref, e65_ref, ats_ref,
                 atd_ref, xl_ref, ad_ref):
    acc = o0_ref[...] + o1_ref[...]
    den = jnp.maximum(acc[:, DEN:DEN + 1], 1e-30)
    h = acc[:, :HID] / den + b_ref[...][None, :]
    h = jnp.maximum(h, 0.0)
    xl = jnp.dot(h, w_ref[...], preferred_element_type=jnp.float32)
    xl = xl + e64_ref[...][None, :]
    as_ = jnp.sum(xl * ats_ref[...][None, :], axis=1)
    ad_ref[...] = jnp.sum(xl * atd_ref[...][None, :], axis=1)
    xl_ref[...] = xl + as_[:, None] * e65_ref[...][None, :]


def _tc_mid(o0, o1, b, w_pad, e64, e65, ats_pad, atd_pad):
    return pl.pallas_call(
        _tc_mid_body,
        grid=(NP_ // _BLK,),
        in_specs=[
            pl.BlockSpec((_BLK, FP), lambda i: (i, 0)),
            pl.BlockSpec((_BLK, FP), lambda i: (i, 0)),
            pl.BlockSpec((HID,), lambda i: (0,)),
            pl.BlockSpec((HID, FP), lambda i: (0, 0)),
            pl.BlockSpec((FP,), lambda i: (0,)),
            pl.BlockSpec((FP,), lambda i: (0,)),
            pl.BlockSpec((FP,), lambda i: (0,)),
            pl.BlockSpec((FP,), lambda i: (0,)),
        ],
        out_specs=[
            pl.BlockSpec((_BLK, FP), lambda i: (i, 0)),
            pl.BlockSpec((_BLK,), lambda i: (i,)),
        ],
        out_shape=[
            jax.ShapeDtypeStruct((NP_, FP), jnp.float32),
            jax.ShapeDtypeStruct((NP_,), jnp.float32),
        ],
    )(o0, o1, b, w_pad, e64, e65, ats_pad, atd_pad)


def _tc_head_body(o0_ref, o1_ref, b_ref, batch_ref, fcw_ref, fcb_ref,
                  out_ref):
    acc = o0_ref[...] + o1_ref[...]
    den = jnp.maximum(acc[:, DEN:DEN + 1], 1e-30)
    h = acc[:, :HID] / den + b_ref[...][None, :]
    gids = lax.broadcasted_iota(jnp.int32, (G, NP_), 0)
    mask = (batch_ref[...][None, :] == gids).astype(jnp.float32)
    sums = jnp.dot(mask, h, preferred_element_type=jnp.float32)
    counts = jnp.sum(mask, axis=1)
    pooled = sums / jnp.maximum(counts, 1.0)[:, None]
    logits = lax.dot_general(pooled, fcw_ref[...], (((1,), (1,)), ((), ())),
                             preferred_element_type=jnp.float32)
    logits = logits + fcb_ref[...][None, :]
    m = jnp.max(logits, axis=1, keepdims=True)
    lse = m + jnp.log(jnp.sum(jnp.exp(logits - m), axis=1, keepdims=True))
    out_ref[...] = logits - lse


def _tc_head(o0, o1, b, batch_p, fcw, fcb):
    return pl.pallas_call(
        _tc_head_body,
        out_shape=jax.ShapeDtypeStruct((G, OUT), jnp.float32),
    )(o0, o1, b, batch_p, fcw, fcb)


def _sc_edge_body(xl_hbm, adst_hbm, srcw_hbm, dstw_hbm,
                  out0_hbm, out1_hbm,
                  out_s,
                  adst_v, src0, dst0, src1, dst1, rows_a, rows_b, ex_v,
                  sem_ga, sem_gb, sem_sa, sem_sb):
    c = lax.axis_index("c")
    s = lax.axis_index("s")
    w = c * NS + s
    r0 = s * STRIPE
    base = w * GPT

    pltpu.sync_copy(adst_hbm, adst_v)

    def _zrow(e, cr):
        for q in range(FP // 16):
            rows_a[e, pl.ds(q * 16, 16)] = jnp.zeros((16,), jnp.float32)
        return cr
    lax.fori_loop(0, 128, _zrow, 0)

    for k in range(STRIPE // 128):
        pltpu.sync_copy(rows_a, out_s.at[pl.ds(r0 + k * 128, 128)])

    def _fetch(g, srcb, dstb, rows, sem_g):
        pltpu.sync_copy(srcw_hbm.at[g], srcb.at[0])
        pltpu.sync_copy(dstw_hbm.at[g], dstb.at[0])
        pltpu.async_copy(xl_hbm.at[srcb.at[0]], rows, sem_g)

    _fetch(base, src0, dst0, rows_a, sem_ga)
    plsc.subcore_barrier()

    def _ex_scale(rows, dstb):
        for j in range(8):
            sl = pl.ds(j * 16, 16)
            e16 = lax.broadcasted_iota(jnp.int32, (16,), 0) + (j * 16)
            c65 = jnp.full((16,), ASRC, jnp.int32)
            a = (plsc.load_gather(rows, [e16, c65])
                 + plsc.load_gather(adst_v, [dstb[0, sl]]))
            a = jnp.where(a >= 0.0, a, a * NEG)
            ex16 = jnp.exp(a)
            ex_v[sl] = ex16
            plsc.store_scatter(rows, [e16, jnp.full((16,), DEN, jnp.int32)],
                               ex16)

        def _scale(j, cr2):
            ex16 = ex_v[pl.ds(j * 16, 16)]
            for t in range(16):
                cf = ex16[t]
                e = j * 16 + t
                for q in range(4):
                    sl2 = pl.ds(q * 16, 16)
                    rows[e, sl2] = rows[e, sl2] * cf
            return cr2
        lax.fori_loop(0, 8, _scale, 0)

    def _pair(i, cr):
        g0 = base + 2 * i
        pltpu.make_async_copy(xl_hbm.at[src0.at[0]], rows_a, sem_ga).wait()
        _ex_scale(rows_a, dst0)
        pltpu.async_copy(rows_a, out_s.at[dst0.at[0]], sem_sa, add=True)

        @pl.when(i > 0)
        def _():
            pltpu.make_async_copy(rows_b, out_s.at[dst1.at[0]], sem_sb).wait()
        _fetch(g0 + 1, src1, dst1, rows_b, sem_gb)

        pltpu.make_async_copy(xl_hbm.at[src1.at[0]], rows_b, sem_gb).wait()
        _ex_scale(rows_b, dst1)
        pltpu.async_copy(rows_b, out_s.at[dst1.at[0]], sem_sb, add=True)

        pltpu.make_async_copy(rows_a, out_s.at[dst0.at[0]], sem_sa).wait()

        @pl.when(i + 1 < GPT // 2)
        def _():
            _fetch(g0 + 2, src0, dst0, rows_a, sem_ga)
        return cr
    lax.fori_loop(0, GPT // 2, _pair, 0)
    pltpu.make_async_copy(rows_b, out_s.at[dst1.at[0]], sem_sb).wait()
    plsc.subcore_barrier()

    @pl.when(c == 0)
    def _():
        for k in range(STRIPE // 128):
            ck = pl.ds(r0 + k * 128, 128)
            pltpu.sync_copy(out_s.at[ck], rows_a)
            pltpu.sync_copy(rows_a, out0_hbm.at[ck])

    @pl.when(c == 1)
    def _():
        for k in range(STRIPE // 128):
            ck = pl.ds(r0 + k * 128, 128)
            pltpu.sync_copy(out_s.at[ck], rows_a)
            pltpu.sync_copy(rows_a, out1_hbm.at[ck])


_sc_edge = functools.partial(
    pl.kernel,
    out_type=(
        jax.ShapeDtypeStruct((NP_, FP), jnp.float32),
        jax.ShapeDtypeStruct((NP_, FP), jnp.float32),
    ),
    mesh=plsc.VectorSubcoreMesh(core_axis_name="c", subcore_axis_name="s"),
    compiler_params=pltpu.CompilerParams(needs_layout_passes=False),
    scratch_types=[
        pltpu.VMEM_SHARED((NP_, FP), jnp.float32),
        pltpu.VMEM((NP_,), jnp.float32),
        pltpu.VMEM((1, 128), jnp.int32),
        pltpu.VMEM((1, 128), jnp.int32),
        pltpu.VMEM((1, 128), jnp.int32),
        pltpu.VMEM((1, 128), jnp.int32),
        pltpu.VMEM((128, FP), jnp.float32),
        pltpu.VMEM((128, FP), jnp.float32),
        pltpu.VMEM((128,), jnp.float32),
        pltpu.SemaphoreType.DMA,
        pltpu.SemaphoreType.DMA,
        pltpu.SemaphoreType.DMA,
        pltpu.SemaphoreType.DMA,
    ],
)(_sc_edge_body)


def kernel(x, edge_index, batch, W1, att_src1, att_dst1, b1,
           W2, att_src2, att_dst2, b2, fc_w, fc_b):
    e = edge_index.shape[1]
    x_p = jnp.pad(x, ((0, NP_ - N), (0, 0)))
    loops = jnp.arange(N, dtype=jnp.int32)
    npad = ETOT - (e + N)
    pad_idx = N + (jnp.arange(npad, dtype=jnp.int32) % PAD_ROWS)
    src = jnp.concatenate([edge_index[0], loops, pad_idx]).reshape(NW * GPT, 128)
    dst = jnp.concatenate([edge_index[1], loops, pad_idx]).reshape(NW * GPT, 128)
    batch_p = jnp.concatenate(
        [batch, jnp.full((NP_ - N,), G, dtype=jnp.int32)])

    e64 = jnp.zeros((FP,), jnp.float32).at[DEN].set(1.0)
    e65 = jnp.zeros((FP,), jnp.float32).at[ASRC].set(1.0)
    w1_pad = jnp.pad(W1, ((0, 0), (0, FP - HID)))
    w2_pad = jnp.pad(W2, ((0, 0), (0, FP - HID)))
    ats1 = jnp.pad(att_src1, (0, FP - HID))
    atd1 = jnp.pad(att_dst1, (0, FP - HID))
    ats2 = jnp.pad(att_src2, (0, FP - HID))
    atd2 = jnp.pad(att_dst2, (0, FP - HID))

    xl1, ad1 = _tc_embed(x_p, w1_pad, e64, e65, ats1, atd1, D)
    o0, o1 = _sc_edge(xl1, ad1, src, dst)
    xl2, ad2 = _tc_mid(o0, o1, b1, w2_pad, e64, e65, ats2, atd2)
    p0, p1 = _sc_edge(xl2, ad2, src, dst)
    return _tc_head(p0, p1, b2, batch_p, fc_w, fc_b)


# speedup vs baseline: 1.2347x; 1.2347x over previous
import functools

import jax
import jax.numpy as jnp
from jax import lax
from jax.experimental import pallas as pl
from jax.experimental.pallas import tpu as pltpu
from jax.experimental.pallas import tpu_sc as plsc

N = 10000
D = 128
HID = 64
OUT = 7
G = 64
NEG = 0.2

FP = 128
DEN = HID
ASRC = HID + 1

NP_ = 10240
STRIPE = NP_ // 16
NC, NS = 2, 16
NW = NC * NS
GPT = 84
CH = 6
NCH = GPT // CH
CHS = 16
EPW = GPT * 128
ETOT = NW * EPW
PAD_ROWS = NP_ - N

_BLK = 1024


def _tc_embed_body(x_ref, w_ref, e64_ref, e65_ref, ats_ref, atd_ref,
                   xl_ref, ad_ref):
    xl = jnp.dot(x_ref[...], w_ref[...], preferred_element_type=jnp.float32)
    xl = xl + e64_ref[...][None, :]
    as_ = jnp.sum(xl * ats_ref[...][None, :], axis=1)
    ad_ref[...] = jnp.sum(xl * atd_ref[...][None, :], axis=1)
    xl_ref[...] = xl + as_[:, None] * e65_ref[...][None, :]


def _tc_embed(x_p, w_pad, e64, e65, ats_pad, atd_pad, d_in):
    return pl.pallas_call(
        _tc_embed_body,
        grid=(NP_ // _BLK,),
        in_specs=[
            pl.BlockSpec((_BLK, d_in), lambda i: (i, 0)),
            pl.BlockSpec((d_in, FP), lambda i: (0, 0)),
            pl.BlockSpec((FP,), lambda i: (0,)),
            pl.BlockSpec((FP,), lambda i: (0,)),
            pl.BlockSpec((FP,), lambda i: (0,)),
            pl.BlockSpec((FP,), lambda i: (0,)),
        ],
        out_specs=[
            pl.BlockSpec((_BLK, FP), lambda i: (i, 0)),
            pl.BlockSpec((_BLK,), lambda i: (i,)),
        ],
        out_shape=[
            jax.ShapeDtypeStruct((NP_, FP), jnp.float32),
            jax.ShapeDtypeStruct((NP_,), jnp.float32),
        ],
    )(x_p, w_pad, e64, e65, ats_pad, atd_pad)


def _tc_mid_body(o0_ref, o1_ref, b_ref, w_ref, e64_ref, e65_ref, ats_ref,
                 atd_ref, xl_ref, ad_ref):
    acc = o0_ref[...] + o1_ref[...]
    den = jnp.maximum(acc[:, DEN:DEN + 1], 1e-30)
    h = acc[:, :HID] / den + b_ref[...][None, :]
    h = jnp.maximum(h, 0.0)
    xl = jnp.dot(h, w_ref[...], preferred_element_type=jnp.float32)
    xl = xl + e64_ref[...][None, :]
    as_ = jnp.sum(xl * ats_ref[...][None, :], axis=1)
    ad_ref[...] = jnp.sum(xl * atd_ref[...][None, :], axis=1)
    xl_ref[...] = xl + as_[:, None] * e65_ref[...][None, :]


def _tc_mid(o0, o1, b, w_pad, e64, e65, ats_pad, atd_pad):
    return pl.pallas_call(
        _tc_mid_body,
        grid=(NP_ // _BLK,),
        in_specs=[
            pl.BlockSpec((_BLK, FP), lambda i: (i, 0)),
            pl.BlockSpec((_BLK, FP), lambda i: (i, 0)),
            pl.BlockSpec((HID,), lambda i: (0,)),
            pl.BlockSpec((HID, FP), lambda i: (0, 0)),
            pl.BlockSpec((FP,), lambda i: (0,)),
            pl.BlockSpec((FP,), lambda i: (0,)),
            pl.BlockSpec((FP,), lambda i: (0,)),
            pl.BlockSpec((FP,), lambda i: (0,)),
        ],
        out_specs=[
            pl.BlockSpec((_BLK, FP), lambda i: (i, 0)),
            pl.BlockSpec((_BLK,), lambda i: (i,)),
        ],
        out_shape=[
            jax.ShapeDtypeStruct((NP_, FP), jnp.float32),
            jax.ShapeDtypeStruct((NP_,), jnp.float32),
        ],
    )(o0, o1, b, w_pad, e64, e65, ats_pad, atd_pad)


def _tc_head_body(o0_ref, o1_ref, b_ref, batch_ref, fcw_ref, fcb_ref,
                  out_ref):
    acc = o0_ref[...] + o1_ref[...]
    den = jnp.maximum(acc[:, DEN:DEN + 1], 1e-30)
    h = acc[:, :HID] / den + b_ref[...][None, :]
    gids = lax.broadcasted_iota(jnp.int32, (G, NP_), 0)
    mask = (batch_ref[...][None, :] == gids).astype(jnp.float32)
    sums = jnp.dot(mask, h, preferred_element_type=jnp.float32)
    counts = jnp.sum(mask, axis=1)
    pooled = sums / jnp.maximum(counts, 1.0)[:, None]
    logits = lax.dot_general(pooled, fcw_ref[...], (((1,), (1,)), ((), ())),
                             preferred_element_type=jnp.float32)
    logits = logits + fcb_ref[...][None, :]
    m = jnp.max(logits, axis=1, keepdims=True)
    lse = m + jnp.log(jnp.sum(jnp.exp(logits - m), axis=1, keepdims=True))
    out_ref[...] = logits - lse


def _tc_head(o0, o1, b, batch_p, fcw, fcb):
    return pl.pallas_call(
        _tc_head_body,
        out_shape=jax.ShapeDtypeStruct((G, OUT), jnp.float32),
    )(o0, o1, b, batch_p, fcw, fcb)


def _sc_edge_body(xl_hbm, adst_hbm, srcdst_hbm,
                  out0_hbm, out1_hbm,
                  out_s,
                  adst_v, idx0, idx1, rows_a, rows_b, ex_v,
                  sem_ga, sem_gb, sem_sa, sem_sb):
    c = lax.axis_index("c")
    s = lax.axis_index("s")
    w = c * NS + s
    r0 = s * STRIPE
    base = w * GPT

    pltpu.sync_copy(adst_hbm, adst_v)

    def _zrow(e, cr):
        for q in range(FP // 16):
            rows_a[e, pl.ds(q * 16, 16)] = jnp.zeros((16,), jnp.float32)
        return cr
    lax.fori_loop(0, 128, _zrow, 0)

    for k in range(STRIPE // 128):
        pltpu.sync_copy(rows_a, out_s.at[pl.ds(r0 + k * 128, 128)])

    kbase = w * NCH
    pltpu.sync_copy(srcdst_hbm.at[pl.ds(kbase * CHS, CHS)], idx0)
    pltpu.async_copy(xl_hbm.at[idx0.at[0]], rows_a, sem_ga)
    plsc.subcore_barrier()

    def _ex_scale(rows, ib, drow):
        for j in range(8):
            sl = pl.ds(j * 16, 16)
            e16 = lax.broadcasted_iota(jnp.int32, (16,), 0) + (j * 16)
            c65 = jnp.full((16,), ASRC, jnp.int32)
            a = (plsc.load_gather(rows, [e16, c65])
                 + plsc.load_gather(adst_v, [ib[drow, sl]]))
            a = jnp.where(a >= 0.0, a, a * NEG)
            ex16 = jnp.exp(a)
            ex_v[sl] = ex16
            plsc.store_scatter(rows, [e16, jnp.full((16,), DEN, jnp.int32)],
                               ex16)

        def _scale(j, cr2):
            ex16 = ex_v[pl.ds(j * 16, 16)]
            for t in range(16):
                cf = ex16[t]
                e = j * 16 + t
                for q in range(4):
                    sl2 = pl.ds(q * 16, 16)
                    rows[e, sl2] = rows[e, sl2] * cf
            return cr2
        lax.fori_loop(0, 8, _scale, 0)

    def _chunk(ib, jb, k, first, last):
        for p in range(CH // 2):
            sA, dA = ib.at[4 * p], ib.at[4 * p + 1]
            sB, dB = ib.at[4 * p + 2], ib.at[4 * p + 3]
            pltpu.make_async_copy(xl_hbm.at[sA], rows_a, sem_ga).wait()
            _ex_scale(rows_a, ib, 4 * p + 1)
            pltpu.async_copy(rows_a, out_s.at[dA], sem_sa, add=True)

            if p == 0:
                if first is None:
                    pltpu.make_async_copy(
                        rows_b, out_s.at[jb.at[2 * CH - 1]], sem_sb).wait()
                else:
                    @pl.when(jnp.logical_not(first))
                    def _():
                        pltpu.make_async_copy(
                            rows_b, out_s.at[jb.at[2 * CH - 1]],
                            sem_sb).wait()
            else:
                pltpu.make_async_copy(
                    rows_b, out_s.at[ib.at[4 * p - 1]], sem_sb).wait()
            pltpu.async_copy(xl_hbm.at[sB], rows_b, sem_gb)

            pltpu.make_async_copy(xl_hbm.at[sB], rows_b, sem_gb).wait()
            _ex_scale(rows_b, ib, 4 * p + 3)
            pltpu.async_copy(rows_b, out_s.at[dB], sem_sb, add=True)

            pltpu.make_async_copy(rows_a, out_s.at[dA], sem_sa).wait()

            if p + 1 < CH // 2:
                pltpu.async_copy(xl_hbm.at[ib.at[4 * p + 4]], rows_a, sem_ga)
            elif last is None:
                pltpu.sync_copy(
                    srcdst_hbm.at[pl.ds((k + 1) * CHS, CHS)], jb)
                pltpu.async_copy(xl_hbm.at[jb.at[0]], rows_a, sem_ga)
            else:
                @pl.when(jnp.logical_not(last))
                def _():
                    pltpu.sync_copy(
                        srcdst_hbm.at[pl.ds((k + 1) * CHS, CHS)], jb)
                    pltpu.async_copy(xl_hbm.at[jb.at[0]], rows_a, sem_ga)

    def _super(ci, cr):
        k0 = kbase + 2 * ci
        _chunk(idx0, idx1, k0, ci == 0, None)
        _chunk(idx1, idx0, k0 + 1, None, 2 * ci + 2 >= NCH)
        return cr
    lax.fori_loop(0, NCH // 2, _super, 0)
    pltpu.make_async_copy(rows_b, out_s.at[idx1.at[2 * CH - 1]],
                          sem_sb).wait()
    plsc.subcore_barrier()

    @pl.when(c == 0)
    def _():
        for k in range(STRIPE // 128):
            ck = pl.ds(r0 + k * 128, 128)
            pltpu.sync_copy(out_s.at[ck], rows_a)
            pltpu.sync_copy(rows_a, out0_hbm.at[ck])

    @pl.when(c == 1)
    def _():
        for k in range(STRIPE // 128):
            ck = pl.ds(r0 + k * 128, 128)
            pltpu.sync_copy(out_s.at[ck], rows_a)
            pltpu.sync_copy(rows_a, out1_hbm.at[ck])


_sc_edge = functools.partial(
    pl.kernel,
    out_type=(
        jax.ShapeDtypeStruct((NP_, FP), jnp.float32),
        jax.ShapeDtypeStruct((NP_, FP), jnp.float32),
    ),
    mesh=plsc.VectorSubcoreMesh(core_axis_name="c", subcore_axis_name="s"),
    compiler_params=pltpu.CompilerParams(needs_layout_passes=False),
    scratch_types=[
        pltpu.VMEM_SHARED((NP_, FP), jnp.float32),
        pltpu.VMEM((NP_,), jnp.float32),
        pltpu.VMEM((CHS, 128), jnp.int32),
        pltpu.VMEM((CHS, 128), jnp.int32),
        pltpu.VMEM((128, FP), jnp.float32),
        pltpu.VMEM((128, FP), jnp.float32),
        pltpu.VMEM((128,), jnp.float32),
        pltpu.SemaphoreType.DMA,
        pltpu.SemaphoreType.DMA,
        pltpu.SemaphoreType.DMA,
        pltpu.SemaphoreType.DMA,
    ],
)(_sc_edge_body)


def kernel(x, edge_index, batch, W1, att_src1, att_dst1, b1,
           W2, att_src2, att_dst2, b2, fc_w, fc_b):
    e = edge_index.shape[1]
    x_p = jnp.pad(x, ((0, NP_ - N), (0, 0)))
    loops = jnp.arange(N, dtype=jnp.int32)
    npad = ETOT - (e + N)
    pad_idx = N + (jnp.arange(npad, dtype=jnp.int32) % PAD_ROWS)
    src = jnp.concatenate([edge_index[0], loops, pad_idx]).reshape(NW * GPT, 128)
    dst = jnp.concatenate([edge_index[1], loops, pad_idx]).reshape(NW * GPT, 128)
    srcdst = jnp.stack([src, dst], axis=1).reshape(NW, NCH, CH * 2, 128)
    srcdst = jnp.pad(srcdst, ((0, 0), (0, 0), (0, CHS - CH * 2), (0, 0)))
    srcdst = srcdst.reshape(NW * NCH * CHS, 128)
    batch_p = jnp.concatenate(
        [batch, jnp.full((NP_ - N,), G, dtype=jnp.int32)])

    e64 = jnp.zeros((FP,), jnp.float32).at[DEN].set(1.0)
    e65 = jnp.zeros((FP,), jnp.float32).at[ASRC].set(1.0)
    w1_pad = jnp.pad(W1, ((0, 0), (0, FP - HID)))
    w2_pad = jnp.pad(W2, ((0, 0), (0, FP - HID)))
    ats1 = jnp.pad(att_src1, (0, FP - HID))
    atd1 = jnp.pad(att_dst1, (0, FP - HID))
    ats2 = jnp.pad(att_src2, (0, FP - HID))
    atd2 = jnp.pad(att_dst2, (0, FP - HID))

    xl1, ad1 = _tc_embed(x_p, w1_pad, e64, e65, ats1, atd1, D)
    o0, o1 = _sc_edge(xl1, ad1, srcdst)
    xl2, ad2 = _tc_mid(o0, o1, b1, w2_pad, e64, e65, ats2, atd2)
    p0, p1 = _sc_edge(xl2, ad2, srcdst)
    return _tc_head(p0, p1, b2, batch_p, fc_w, fc_b)
